# baseline (speedup 1.0000x reference)
import functools

import jax
import jax.numpy as jnp
from jax import lax
from jax.experimental import pallas as pl
from jax.experimental.pallas import tpu as pltpu, tpu_sc as plsc

_B, _N = 64, 512 * 512
_K = int(0.75 * _N)
_CH = 16384
_NCHUNK = _N // _CH
_NBINS1 = 8192
_NBINS2 = 32768


def _scan_raw13(hist, target, lane):
    def chunk_body(i, carry):
        c = jnp.where(i < 256, 255 - i, i)
        b_chunk, above, running = carry
        s = jnp.sum(hist[pl.ds(c * 16, 16)])
        crossed = (running < target) & (running + s >= target)
        b_chunk = jnp.where(crossed, c, b_chunk)
        above = jnp.where(crossed, running, above)
        return (b_chunk, above, running + s)

    z = jnp.int32(0)
    b_chunk, above0, _ = lax.fori_loop(0, 512, chunk_body, (z, z, z))

    v = hist[pl.ds(b_chunk * 16, 16)]
    neg = b_chunk >= 256
    suffix = lax.rev(plsc.cumsum(lax.rev(v, (0,))), (0,))
    prefix = plsc.cumsum(v)
    jp1 = plsc.all_reduce_population_count((above0 + suffix) >= target)
    b_pos = b_chunk * 16 + jp1 - 1
    ab_pos = above0 + jnp.sum(jnp.where(lane >= jp1, v, 0))
    jn = 16 - plsc.all_reduce_population_count((above0 + prefix) >= target)
    b_neg = b_chunk * 16 + jn
    ab_neg = above0 + jnp.sum(jnp.where(lane < jn, v, 0))
    b = jnp.where(neg, b_neg, b_pos)
    above = jnp.where(neg, ab_neg, ab_pos)
    return b, above, neg


def _scan_asc(hist, nbins, target, lane):
    nch = nbins // 16

    def chunk_body(i, carry):
        c = nch - 1 - i
        b_chunk, above, running = carry
        s = jnp.sum(hist[pl.ds(c * 16, 16)])
        crossed = (running < target) & (running + s >= target)
        b_chunk = jnp.where(crossed, c, b_chunk)
        above = jnp.where(crossed, running, above)
        return (b_chunk, above, running + s)

    z = jnp.int32(0)
    b_chunk, above0, _ = lax.fori_loop(0, nch, chunk_body, (z, z, z))

    v = hist[pl.ds(b_chunk * 16, 16)]
    suffix = lax.rev(plsc.cumsum(lax.rev(v, (0,))), (0,))
    jp1 = plsc.all_reduce_population_count((above0 + suffix) >= target)
    bv = b_chunk * 16 + jp1 - 1
    above = above0 + jnp.sum(jnp.where(lane >= jp1, v, 0))
    return bv, above


def _make_threshold_kernel():
    mesh = plsc.VectorSubcoreMesh(core_axis_name="c", subcore_axis_name="s")

    @functools.partial(
        pl.kernel,
        out_type=jax.ShapeDtypeStruct((_B, 16), jnp.float32),
        mesh=mesh,
        compiler_params=pltpu.CompilerParams(needs_layout_passes=False),
        scratch_types=[
            pltpu.VMEM((_CH,), jnp.float32),
            pltpu.VMEM((_CH,), jnp.float32),
            pltpu.VMEM((_NBINS2,), jnp.int32),
            pltpu.VMEM((16,), jnp.float32),
            pltpu.SemaphoreType.DMA,
            pltpu.SemaphoreType.DMA,
        ],
    )
    def thr_kernel(x_hbm, out_hbm, bufa, bufb, hist, thrb, sema, semb):
        wid = lax.axis_index("s") * 2 + lax.axis_index("c")
        lane = lax.iota(jnp.int32, 16)
        ones16 = jnp.ones((16,), jnp.int32)
        zero16 = jnp.zeros((16,), jnp.int32)
        bufs = (bufa, bufb)
        sems = (sema, semb)

        def zbody(i, _):
            base = i * 64
            for u in range(4):
                hist[pl.ds(base + u * 16, 16)] = zero16
            return 0

        def stream(body):
            cps = [pltpu.async_copy(x_hbm.at[row, pl.ds(0, _CH)], bufa, sema), None]
            for c in range(_NCHUNK):
                cur = c % 2
                nxt = 1 - cur
                if c + 1 < _NCHUNK:
                    cps[nxt] = pltpu.async_copy(
                        x_hbm.at[row, pl.ds((c + 1) * _CH, _CH)], bufs[nxt], sems[nxt]
                    )
                cps[cur].wait()
                lax.fori_loop(0, _CH // 64, functools.partial(body, bufs[cur]), 0)

        for r in range(2):
            row = wid * 2 + r

            lax.fori_loop(0, _NBINS1 // 64, zbody, 0)

            def p1body(buf, i, _):
                base = i * 64
                for u in range(4):
                    bits = plsc.bitcast(buf[pl.ds(base + u * 16, 16)], jnp.uint32)
                    b = plsc.bitcast(bits >> 19, jnp.int32)
                    plsc.addupdate_scatter(hist, [b], ones16)
                return 0

            stream(p1body)
            b1, above1, neg = _scan_raw13(hist, jnp.int32(_K), lane)
            k2 = jnp.int32(_K) - above1
            b1u = plsc.bitcast(b1, jnp.uint32)
            negmask = jnp.where(
                neg,
                jnp.full((16,), 0x7FFFF, jnp.uint32),
                jnp.zeros((16,), jnp.uint32),
            )

            lax.fori_loop(0, _NBINS2 // 64, zbody, 0)

            def p2body(buf, i, _):
                base = i * 64
                for u in range(4):
                    bits = plsc.bitcast(buf[pl.ds(base + u * 16, 16)], jnp.uint32)
                    m = (bits >> 19) == b1u
                    f = ((bits ^ negmask) >> 4) & jnp.uint32(0x7FFF)
                    plsc.addupdate_scatter(
                        hist, [plsc.bitcast(f, jnp.int32)], ones16, mask=m
                    )
                return 0

            stream(p2body)
            b2, _ = _scan_asc(hist, _NBINS2, k2, lane)

            b2u = plsc.bitcast(b2, jnp.uint32)
            fbits = (b1u << 19) | (((b2u << 4) ^ negmask) & jnp.uint32(0x7FFFF))
            thrb[...] = plsc.bitcast(fbits, jnp.float32)
            pltpu.sync_copy(thrb, out_hbm.at[row])

    return thr_kernel


def _mask_body(x_ref, thr_ref, out_ref):
    t = thr_ref[0, 0, 0]
    out_ref[0, 0] = (x_ref[0] >= t).astype(jnp.float32)


def kernel(importance):
    b, h, w = importance.shape
    k = max(1, int(0.75 * h * w))
    thr = _make_threshold_kernel()(importance.reshape(b, h * w))
    mask = pl.pallas_call(
        _mask_body,
        grid=(b,),
        in_specs=[
            pl.BlockSpec((1, h, w), lambda i: (i, 0, 0)),
            pl.BlockSpec((1, 1, 16), lambda i: (i, 0, 0)),
        ],
        out_specs=pl.BlockSpec((1, 1, h, w), lambda i: (i, 0, 0, 0)),
        out_shape=jax.ShapeDtypeStruct((b, 1, h, w), jnp.float32),
    )(importance, thr.reshape(b, 1, 16))
    return (mask, jnp.float32(k / (h * w)))


# speedup vs baseline: 1.0146x; 1.0146x over previous
import functools

import jax
import jax.numpy as jnp
from jax import lax
from jax.experimental import pallas as pl
from jax.experimental.pallas import tpu as pltpu, tpu_sc as plsc

_B, _N = 64, 512 * 512
_K = int(0.75 * _N)
_CH = 32768
_NCHUNK = _N // _CH
_NBINS1 = 8192
_NBINS2 = 32768


def _scan_raw13(hist, target, lane):
    def chunk_body(i, carry):
        c = jnp.where(i < 256, 255 - i, i)
        b_chunk, above, running = carry
        s = jnp.sum(hist[pl.ds(c * 16, 16)])
        crossed = (running < target) & (running + s >= target)
        b_chunk = jnp.where(crossed, c, b_chunk)
        above = jnp.where(crossed, running, above)
        return (b_chunk, above, running + s)

    z = jnp.int32(0)
    b_chunk, above0, _ = lax.fori_loop(0, 512, chunk_body, (z, z, z))

    v = hist[pl.ds(b_chunk * 16, 16)]
    neg = b_chunk >= 256
    suffix = lax.rev(plsc.cumsum(lax.rev(v, (0,))), (0,))
    prefix = plsc.cumsum(v)
    jp1 = plsc.all_reduce_population_count((above0 + suffix) >= target)
    b_pos = b_chunk * 16 + jp1 - 1
    ab_pos = above0 + jnp.sum(jnp.where(lane >= jp1, v, 0))
    jn = 16 - plsc.all_reduce_population_count((above0 + prefix) >= target)
    b_neg = b_chunk * 16 + jn
    ab_neg = above0 + jnp.sum(jnp.where(lane < jn, v, 0))
    b = jnp.where(neg, b_neg, b_pos)
    above = jnp.where(neg, ab_neg, ab_pos)
    return b, above, neg


def _scan_asc(hist, nbins, target, lane):
    nch = nbins // 16

    def chunk_body(i, carry):
        c = nch - 1 - i
        b_chunk, above, running = carry
        s = jnp.sum(hist[pl.ds(c * 16, 16)])
        crossed = (running < target) & (running + s >= target)
        b_chunk = jnp.where(crossed, c, b_chunk)
        above = jnp.where(crossed, running, above)
        return (b_chunk, above, running + s)

    z = jnp.int32(0)
    b_chunk, above0, _ = lax.fori_loop(0, nch, chunk_body, (z, z, z))

    v = hist[pl.ds(b_chunk * 16, 16)]
    suffix = lax.rev(plsc.cumsum(lax.rev(v, (0,))), (0,))
    jp1 = plsc.all_reduce_population_count((above0 + suffix) >= target)
    bv = b_chunk * 16 + jp1 - 1
    above = above0 + jnp.sum(jnp.where(lane >= jp1, v, 0))
    return bv, above


def _make_threshold_kernel():
    mesh = plsc.VectorSubcoreMesh(core_axis_name="c", subcore_axis_name="s")

    @functools.partial(
        pl.kernel,
        out_type=jax.ShapeDtypeStruct((_B, 16), jnp.float32),
        mesh=mesh,
        compiler_params=pltpu.CompilerParams(needs_layout_passes=False),
        scratch_types=[
            pltpu.VMEM((_CH,), jnp.float32),
            pltpu.VMEM((_CH,), jnp.float32),
            pltpu.VMEM((_NBINS2,), jnp.int32),
            pltpu.VMEM((16,), jnp.float32),
            pltpu.SemaphoreType.DMA,
            pltpu.SemaphoreType.DMA,
        ],
    )
    def thr_kernel(x_hbm, out_hbm, bufa, bufb, hist, thrb, sema, semb):
        wid = lax.axis_index("s") * 2 + lax.axis_index("c")
        lane = lax.iota(jnp.int32, 16)
        ones16 = jnp.ones((16,), jnp.int32)
        zero16 = jnp.zeros((16,), jnp.int32)
        bufs = (bufa, bufb)
        sems = (sema, semb)

        def zbody(i, _):
            base = i * 64
            for u in range(4):
                hist[pl.ds(base + u * 16, 16)] = zero16
            return 0

        def stream(body):
            cps = [pltpu.async_copy(x_hbm.at[row, pl.ds(0, _CH)], bufa, sema), None]
            for c in range(_NCHUNK):
                cur = c % 2
                nxt = 1 - cur
                if c + 1 < _NCHUNK:
                    cps[nxt] = pltpu.async_copy(
                        x_hbm.at[row, pl.ds((c + 1) * _CH, _CH)], bufs[nxt], sems[nxt]
                    )
                cps[cur].wait()
                lax.fori_loop(0, _CH // 128, functools.partial(body, bufs[cur]), 0)

        for r in range(2):
            row = wid * 2 + r

            lax.fori_loop(0, _NBINS1 // 64, zbody, 0)

            def p1body(buf, i, _):
                base = i * 128
                for u in range(8):
                    bits = plsc.bitcast(buf[pl.ds(base + u * 16, 16)], jnp.uint32)
                    b = plsc.bitcast(bits >> 19, jnp.int32)
                    plsc.addupdate_scatter(hist, [b], ones16)
                return 0

            stream(p1body)
            b1, above1, neg = _scan_raw13(hist, jnp.int32(_K), lane)
            k2 = jnp.int32(_K) - above1
            b1u = plsc.bitcast(b1, jnp.uint32)
            negmask = jnp.where(
                neg,
                jnp.full((16,), 0x7FFFF, jnp.uint32),
                jnp.zeros((16,), jnp.uint32),
            )

            lax.fori_loop(0, _NBINS2 // 64, zbody, 0)

            def p2body(buf, i, _):
                base = i * 128
                for u in range(8):
                    bits = plsc.bitcast(buf[pl.ds(base + u * 16, 16)], jnp.uint32)
                    m = (bits >> 19) == b1u
                    f = ((bits ^ negmask) >> 4) & jnp.uint32(0x7FFF)
                    plsc.addupdate_scatter(
                        hist, [plsc.bitcast(f, jnp.int32)], ones16, mask=m
                    )
                return 0

            stream(p2body)
            b2, _ = _scan_asc(hist, _NBINS2, k2, lane)

            b2u = plsc.bitcast(b2, jnp.uint32)
            fbits = (b1u << 19) | (((b2u << 4) ^ negmask) & jnp.uint32(0x7FFFF))
            thrb[...] = plsc.bitcast(fbits, jnp.float32)
            pltpu.sync_copy(thrb, out_hbm.at[row])

    return thr_kernel


def _mask_body(x_ref, thr_ref, out_ref):
    t = thr_ref[0, 0, 0]
    out_ref[0, 0] = (x_ref[0] >= t).astype(jnp.float32)


def kernel(importance):
    b, h, w = importance.shape
    k = max(1, int(0.75 * h * w))
    thr = _make_threshold_kernel()(importance.reshape(b, h * w))
    mask = pl.pallas_call(
        _mask_body,
        grid=(b,),
        in_specs=[
            pl.BlockSpec((1, h, w), lambda i: (i, 0, 0)),
            pl.BlockSpec((1, 1, 16), lambda i: (i, 0, 0)),
        ],
        out_specs=pl.BlockSpec((1, 1, h, w), lambda i: (i, 0, 0, 0)),
        out_shape=jax.ShapeDtypeStruct((b, 1, h, w), jnp.float32),
    )(importance, thr.reshape(b, 1, 16))
    return (mask, jnp.float32(k / (h * w)))


# speedup vs baseline: 2.3338x; 2.3002x over previous
import functools

import jax
import jax.numpy as jnp
from jax import lax
from jax.experimental import pallas as pl
from jax.experimental.pallas import tpu as pltpu, tpu_sc as plsc

_B, _N = 64, 512 * 512
_K = int(0.75 * _N)
_CH = 32768
_NCHUNK = _N // _CH
_NBINS1 = 8192
_NBINS2 = 32768


def _scan_raw13(hist, target, lane):
    def chunk_body(i, carry):
        c = jnp.where(i < 256, 255 - i, i)
        b_chunk, above, running = carry
        s = jnp.sum(hist[pl.ds(c * 16, 16)])
        crossed = (running < target) & (running + s >= target)
        b_chunk = jnp.where(crossed, c, b_chunk)
        above = jnp.where(crossed, running, above)
        return (b_chunk, above, running + s)

    z = jnp.int32(0)
    b_chunk, above0, _ = lax.fori_loop(0, 512, chunk_body, (z, z, z))

    v = hist[pl.ds(b_chunk * 16, 16)]
    neg = b_chunk >= 256
    suffix = lax.rev(plsc.cumsum(lax.rev(v, (0,))), (0,))
    prefix = plsc.cumsum(v)
    jp1 = plsc.all_reduce_population_count((above0 + suffix) >= target)
    b_pos = b_chunk * 16 + jp1 - 1
    ab_pos = above0 + jnp.sum(jnp.where(lane >= jp1, v, 0))
    jn = 16 - plsc.all_reduce_population_count((above0 + prefix) >= target)
    b_neg = b_chunk * 16 + jn
    ab_neg = above0 + jnp.sum(jnp.where(lane < jn, v, 0))
    b = jnp.where(neg, b_neg, b_pos)
    above = jnp.where(neg, ab_neg, ab_pos)
    return b, above, neg


def _scan_asc(hist, nbins, target, lane):
    nch = nbins // 16

    def chunk_body(i, carry):
        c = nch - 1 - i
        b_chunk, above, running = carry
        s = jnp.sum(hist[pl.ds(c * 16, 16)])
        crossed = (running < target) & (running + s >= target)
        b_chunk = jnp.where(crossed, c, b_chunk)
        above = jnp.where(crossed, running, above)
        return (b_chunk, above, running + s)

    z = jnp.int32(0)
    b_chunk, above0, _ = lax.fori_loop(0, nch, chunk_body, (z, z, z))

    v = hist[pl.ds(b_chunk * 16, 16)]
    suffix = lax.rev(plsc.cumsum(lax.rev(v, (0,))), (0,))
    jp1 = plsc.all_reduce_population_count((above0 + suffix) >= target)
    bv = b_chunk * 16 + jp1 - 1
    above = above0 + jnp.sum(jnp.where(lane >= jp1, v, 0))
    return bv, above


def _make_threshold_kernel():
    mesh = plsc.VectorSubcoreMesh(core_axis_name="c", subcore_axis_name="s")

    @functools.partial(
        pl.kernel,
        out_type=jax.ShapeDtypeStruct((_B, 16), jnp.float32),
        mesh=mesh,
        compiler_params=pltpu.CompilerParams(needs_layout_passes=False),
        scratch_types=[
            pltpu.VMEM((_CH,), jnp.float32),
            pltpu.VMEM((_CH,), jnp.float32),
            pltpu.VMEM((_NBINS2,), jnp.int32),
            pltpu.VMEM((16,), jnp.float32),
            pltpu.SemaphoreType.DMA,
            pltpu.SemaphoreType.DMA,
        ],
    )
    def thr_kernel(x_hbm, out_hbm, bufa, bufb, hist, thrb, sema, semb):
        wid = lax.axis_index("s") * 2 + lax.axis_index("c")
        lane = lax.iota(jnp.int32, 16)
        ones16 = jnp.ones((16,), jnp.int32)
        zero16 = jnp.zeros((16,), jnp.int32)
        bufs = (bufa, bufb)
        sems = (sema, semb)

        def zbody(i, _):
            base = i * 64
            for u in range(4):
                hist[pl.ds(base + u * 16, 16)] = zero16
            return 0

        def stream(body):
            cps = [pltpu.async_copy(x_hbm.at[row, pl.ds(0, _CH)], bufa, sema), None]
            for c in range(_NCHUNK):
                cur = c % 2
                nxt = 1 - cur
                if c + 1 < _NCHUNK:
                    cps[nxt] = pltpu.async_copy(
                        x_hbm.at[row, pl.ds((c + 1) * _CH, _CH)], bufs[nxt], sems[nxt]
                    )
                cps[cur].wait()
                lax.fori_loop(0, _CH // 128, functools.partial(body, bufs[cur]), 0)

        for r in range(2):
            row = wid * 2 + r

            lax.fori_loop(0, _NBINS1 // 64, zbody, 0)

            def p1body(buf, i, _):
                base = i * 128
                bs = []
                for u in range(8):
                    bits = plsc.bitcast(buf[pl.ds(base + u * 16, 16)], jnp.uint32)
                    bs.append(plsc.bitcast(bits >> 19, jnp.int32))
                for b in bs:
                    plsc.addupdate_scatter(hist, [b], ones16)
                return 0

            stream(p1body)
            b1, above1, neg = _scan_raw13(hist, jnp.int32(_K), lane)
            k2 = jnp.int32(_K) - above1
            b1u = plsc.bitcast(b1, jnp.uint32)
            negmask = jnp.where(
                neg,
                jnp.full((16,), 0x7FFFF, jnp.uint32),
                jnp.zeros((16,), jnp.uint32),
            )

            lax.fori_loop(0, _NBINS2 // 64, zbody, 0)

            def p2body(buf, i, _):
                base = i * 128
                mf = []
                for u in range(8):
                    bits = plsc.bitcast(buf[pl.ds(base + u * 16, 16)], jnp.uint32)
                    m = (bits >> 19) == b1u
                    f = ((bits ^ negmask) >> 4) & jnp.uint32(0x7FFF)
                    mf.append((m, plsc.bitcast(f, jnp.int32)))
                for m, f in mf:
                    plsc.addupdate_scatter(hist, [f], ones16, mask=m)
                return 0

            stream(p2body)
            b2, _ = _scan_asc(hist, _NBINS2, k2, lane)

            b2u = plsc.bitcast(b2, jnp.uint32)
            fbits = (b1u << 19) | (((b2u << 4) ^ negmask) & jnp.uint32(0x7FFFF))
            thrb[...] = plsc.bitcast(fbits, jnp.float32)
            pltpu.sync_copy(thrb, out_hbm.at[row])

    return thr_kernel


def _mask_body(x_ref, thr_ref, out_ref):
    t = thr_ref[0, 0, 0]
    out_ref[0, 0] = (x_ref[0] >= t).astype(jnp.float32)


def kernel(importance):
    b, h, w = importance.shape
    k = max(1, int(0.75 * h * w))
    thr = _make_threshold_kernel()(importance.reshape(b, h * w))
    mask = pl.pallas_call(
        _mask_body,
        grid=(b,),
        in_specs=[
            pl.BlockSpec((1, h, w), lambda i: (i, 0, 0)),
            pl.BlockSpec((1, 1, 16), lambda i: (i, 0, 0)),
        ],
        out_specs=pl.BlockSpec((1, 1, h, w), lambda i: (i, 0, 0, 0)),
        out_shape=jax.ShapeDtypeStruct((b, 1, h, w), jnp.float32),
    )(importance, thr.reshape(b, 1, 16))
    return (mask, jnp.float32(k / (h * w)))


# speedup vs baseline: 2.4628x; 1.0552x over previous
import functools

import jax
import jax.numpy as jnp
from jax import lax
from jax.experimental import pallas as pl
from jax.experimental.pallas import tpu as pltpu, tpu_sc as plsc

_B, _N = 64, 512 * 512
_K = int(0.75 * _N)
_CH = 32768
_NCHUNK = _N // _CH
_NBINS1 = 8192
_NBINS2 = 8192


def _scan_raw13(hist, target, lane):
    def chunk_body(i, carry):
        c = jnp.where(i < 256, 255 - i, i)
        b_chunk, above, running = carry
        s = jnp.sum(hist[pl.ds(c * 16, 16)])
        crossed = (running < target) & (running + s >= target)
        b_chunk = jnp.where(crossed, c, b_chunk)
        above = jnp.where(crossed, running, above)
        return (b_chunk, above, running + s)

    z = jnp.int32(0)
    b_chunk, above0, _ = lax.fori_loop(0, 512, chunk_body, (z, z, z))

    v = hist[pl.ds(b_chunk * 16, 16)]
    neg = b_chunk >= 256
    suffix = lax.rev(plsc.cumsum(lax.rev(v, (0,))), (0,))
    prefix = plsc.cumsum(v)
    jp1 = plsc.all_reduce_population_count((above0 + suffix) >= target)
    b_pos = b_chunk * 16 + jp1 - 1
    ab_pos = above0 + jnp.sum(jnp.where(lane >= jp1, v, 0))
    jn = 16 - plsc.all_reduce_population_count((above0 + prefix) >= target)
    b_neg = b_chunk * 16 + jn
    ab_neg = above0 + jnp.sum(jnp.where(lane < jn, v, 0))
    b = jnp.where(neg, b_neg, b_pos)
    above = jnp.where(neg, ab_neg, ab_pos)
    return b, above, neg


def _scan_asc(hist, nbins, target, lane):
    nch = nbins // 16

    def chunk_body(i, carry):
        c = nch - 1 - i
        b_chunk, above, running = carry
        s = jnp.sum(hist[pl.ds(c * 16, 16)])
        crossed = (running < target) & (running + s >= target)
        b_chunk = jnp.where(crossed, c, b_chunk)
        above = jnp.where(crossed, running, above)
        return (b_chunk, above, running + s)

    z = jnp.int32(0)
    b_chunk, above0, _ = lax.fori_loop(0, nch, chunk_body, (z, z, z))

    v = hist[pl.ds(b_chunk * 16, 16)]
    suffix = lax.rev(plsc.cumsum(lax.rev(v, (0,))), (0,))
    jp1 = plsc.all_reduce_population_count((above0 + suffix) >= target)
    bv = b_chunk * 16 + jp1 - 1
    above = above0 + jnp.sum(jnp.where(lane >= jp1, v, 0))
    return bv, above


def _make_threshold_kernel():
    mesh = plsc.VectorSubcoreMesh(core_axis_name="c", subcore_axis_name="s")

    @functools.partial(
        pl.kernel,
        out_type=jax.ShapeDtypeStruct((_B, 16), jnp.float32),
        mesh=mesh,
        compiler_params=pltpu.CompilerParams(needs_layout_passes=False),
        scratch_types=[
            pltpu.VMEM((_CH,), jnp.float32),
            pltpu.VMEM((_CH,), jnp.float32),
            pltpu.VMEM((_NBINS2,), jnp.int32),
            pltpu.VMEM((16,), jnp.float32),
            pltpu.SemaphoreType.DMA,
            pltpu.SemaphoreType.DMA,
        ],
    )
    def thr_kernel(x_hbm, out_hbm, bufa, bufb, hist, thrb, sema, semb):
        wid = lax.axis_index("s") * 2 + lax.axis_index("c")
        lane = lax.iota(jnp.int32, 16)
        ones16 = jnp.ones((16,), jnp.int32)
        zero16 = jnp.zeros((16,), jnp.int32)
        bufs = (bufa, bufb)
        sems = (sema, semb)

        def zbody(i, _):
            base = i * 64
            for u in range(4):
                hist[pl.ds(base + u * 16, 16)] = zero16
            return 0

        def stream(body):
            cps = [pltpu.async_copy(x_hbm.at[row, pl.ds(0, _CH)], bufa, sema), None]
            for c in range(_NCHUNK):
                cur = c % 2
                nxt = 1 - cur
                if c + 1 < _NCHUNK:
                    cps[nxt] = pltpu.async_copy(
                        x_hbm.at[row, pl.ds((c + 1) * _CH, _CH)], bufs[nxt], sems[nxt]
                    )
                cps[cur].wait()
                lax.fori_loop(0, _CH // 128, functools.partial(body, bufs[cur]), 0)

        for r in range(2):
            row = wid * 2 + r

            lax.fori_loop(0, _NBINS1 // 64, zbody, 0)

            def p1body(buf, i, _):
                base = i * 128
                bs = []
                for u in range(8):
                    bits = plsc.bitcast(buf[pl.ds(base + u * 16, 16)], jnp.uint32)
                    bs.append(plsc.bitcast(bits >> 19, jnp.int32))
                for b in bs:
                    plsc.addupdate_scatter(hist, [b], ones16)
                return 0

            stream(p1body)
            b1, above1, neg = _scan_raw13(hist, jnp.int32(_K), lane)
            k2 = jnp.int32(_K) - above1
            b1u = plsc.bitcast(b1, jnp.uint32)
            negmask = jnp.where(
                neg,
                jnp.full((16,), 0x7FFFF, jnp.uint32),
                jnp.zeros((16,), jnp.uint32),
            )

            lax.fori_loop(0, _NBINS2 // 64, zbody, 0)

            def p2body(buf, i, _):
                base = i * 128
                mf = []
                for u in range(8):
                    bits = plsc.bitcast(buf[pl.ds(base + u * 16, 16)], jnp.uint32)
                    m = (bits >> 19) == b1u
                    f = ((bits ^ negmask) >> 6) & jnp.uint32(0x1FFF)
                    mf.append((m, plsc.bitcast(f, jnp.int32)))
                for m, f in mf:
                    plsc.addupdate_scatter(hist, [f], ones16, mask=m)
                return 0

            stream(p2body)
            b2, _ = _scan_asc(hist, _NBINS2, k2, lane)

            b2u = plsc.bitcast(b2, jnp.uint32)
            fbits = (b1u << 19) | (((b2u << 6) ^ negmask) & jnp.uint32(0x7FFFF))
            thrb[...] = plsc.bitcast(fbits, jnp.float32)
            pltpu.sync_copy(thrb, out_hbm.at[row])

    return thr_kernel


def _mask_body(x_ref, thr_ref, out_ref):
    t = thr_ref[0, 0, 0]
    out_ref[0, 0] = (x_ref[0] >= t).astype(jnp.float32)


def kernel(importance):
    b, h, w = importance.shape
    k = max(1, int(0.75 * h * w))
    thr = _make_threshold_kernel()(importance.reshape(b, h * w))
    mask = pl.pallas_call(
        _mask_body,
        grid=(b,),
        in_specs=[
            pl.BlockSpec((1, h, w), lambda i: (i, 0, 0)),
            pl.BlockSpec((1, 1, 16), lambda i: (i, 0, 0)),
        ],
        out_specs=pl.BlockSpec((1, 1, h, w), lambda i: (i, 0, 0, 0)),
        out_shape=jax.ShapeDtypeStruct((b, 1, h, w), jnp.float32),
    )(importance, thr.reshape(b, 1, 16))
    return (mask, jnp.float32(k / (h * w)))


# speedup vs baseline: 2.6302x; 1.0680x over previous
import functools

import jax
import jax.numpy as jnp
from jax import lax
from jax.experimental import pallas as pl
from jax.experimental.pallas import tpu as pltpu, tpu_sc as plsc

_B, _N = 64, 512 * 512
_K = int(0.75 * _N)
_CH = 32768
_NCHUNK = _N // _CH
_NBINS1 = 8192
_NBINS2 = 8192


def _scan_raw13(hist, target, lane):
    def chunk_body(i, carry):
        c = jnp.where(i < 256, 255 - i, i)
        b_chunk, above, running = carry
        s = jnp.sum(hist[pl.ds(c * 16, 16)])
        crossed = (running < target) & (running + s >= target)
        b_chunk = jnp.where(crossed, c, b_chunk)
        above = jnp.where(crossed, running, above)
        return (b_chunk, above, running + s)

    z = jnp.int32(0)
    b_chunk, above0, _ = lax.fori_loop(0, 512, chunk_body, (z, z, z))

    v = hist[pl.ds(b_chunk * 16, 16)]
    neg = b_chunk >= 256
    suffix = lax.rev(plsc.cumsum(lax.rev(v, (0,))), (0,))
    prefix = plsc.cumsum(v)
    jp1 = plsc.all_reduce_population_count((above0 + suffix) >= target)
    b_pos = b_chunk * 16 + jp1 - 1
    ab_pos = above0 + jnp.sum(jnp.where(lane >= jp1, v, 0))
    jn = 16 - plsc.all_reduce_population_count((above0 + prefix) >= target)
    b_neg = b_chunk * 16 + jn
    ab_neg = above0 + jnp.sum(jnp.where(lane < jn, v, 0))
    b = jnp.where(neg, b_neg, b_pos)
    above = jnp.where(neg, ab_neg, ab_pos)
    return b, above, neg


def _scan_asc(hist, nbins, target, lane):
    nch = nbins // 16

    def chunk_body(i, carry):
        c = nch - 1 - i
        b_chunk, above, running = carry
        s = jnp.sum(hist[pl.ds(c * 16, 16)])
        crossed = (running < target) & (running + s >= target)
        b_chunk = jnp.where(crossed, c, b_chunk)
        above = jnp.where(crossed, running, above)
        return (b_chunk, above, running + s)

    z = jnp.int32(0)
    b_chunk, above0, _ = lax.fori_loop(0, nch, chunk_body, (z, z, z))

    v = hist[pl.ds(b_chunk * 16, 16)]
    suffix = lax.rev(plsc.cumsum(lax.rev(v, (0,))), (0,))
    jp1 = plsc.all_reduce_population_count((above0 + suffix) >= target)
    bv = b_chunk * 16 + jp1 - 1
    above = above0 + jnp.sum(jnp.where(lane >= jp1, v, 0))
    return bv, above


def _make_threshold_kernel():
    mesh = plsc.VectorSubcoreMesh(core_axis_name="c", subcore_axis_name="s")

    @functools.partial(
        pl.kernel,
        out_type=jax.ShapeDtypeStruct((_B, 16), jnp.float32),
        mesh=mesh,
        compiler_params=pltpu.CompilerParams(needs_layout_passes=False),
        scratch_types=[
            pltpu.VMEM((_CH,), jnp.float32),
            pltpu.VMEM((_CH,), jnp.float32),
            pltpu.VMEM((_NBINS2,), jnp.int32),
            pltpu.VMEM((16,), jnp.float32),
            pltpu.SemaphoreType.DMA,
            pltpu.SemaphoreType.DMA,
        ],
    )
    def thr_kernel(x_hbm, out_hbm, bufa, bufb, hist, thrb, sema, semb):
        wid = lax.axis_index("s") * 2 + lax.axis_index("c")
        lane = lax.iota(jnp.int32, 16)
        ones16 = jnp.ones((16,), jnp.int32)
        zero16 = jnp.zeros((16,), jnp.int32)
        bufs = (bufa, bufb)
        sems = (sema, semb)

        def zbody(i, _):
            base = i * 64
            for u in range(4):
                hist[pl.ds(base + u * 16, 16)] = zero16
            return 0

        def stream(body):
            cps = [pltpu.async_copy(x_hbm.at[row, pl.ds(0, _CH)], bufa, sema), None]
            for c in range(_NCHUNK):
                cur = c % 2
                nxt = 1 - cur
                if c + 1 < _NCHUNK:
                    cps[nxt] = pltpu.async_copy(
                        x_hbm.at[row, pl.ds((c + 1) * _CH, _CH)], bufs[nxt], sems[nxt]
                    )
                cps[cur].wait()
                lax.fori_loop(0, _CH // 256, functools.partial(body, bufs[cur]), 0)

        for r in range(2):
            row = wid * 2 + r

            lax.fori_loop(0, _NBINS1 // 64, zbody, 0)

            def p1body(buf, i, _):
                base = i * 256
                bs = []
                for u in range(16):
                    bits = plsc.bitcast(buf[pl.ds(base + u * 16, 16)], jnp.uint32)
                    bs.append(plsc.bitcast(bits >> 19, jnp.int32))
                for b in bs:
                    plsc.addupdate_scatter(hist, [b], ones16)
                return 0

            stream(p1body)
            b1, above1, neg = _scan_raw13(hist, jnp.int32(_K), lane)
            k2 = jnp.int32(_K) - above1
            b1u = plsc.bitcast(b1, jnp.uint32)
            negmask = jnp.where(
                neg,
                jnp.full((16,), 0x7FFFF, jnp.uint32),
                jnp.zeros((16,), jnp.uint32),
            )

            lax.fori_loop(0, _NBINS2 // 64, zbody, 0)

            def p2body(buf, i, _):
                base = i * 256
                mf = []
                for u in range(16):
                    bits = plsc.bitcast(buf[pl.ds(base + u * 16, 16)], jnp.uint32)
                    m = (bits >> 19) == b1u
                    f = ((bits ^ negmask) >> 6) & jnp.uint32(0x1FFF)
                    mf.append((m, plsc.bitcast(f, jnp.int32)))
                for m, f in mf:
                    plsc.addupdate_scatter(hist, [f], ones16, mask=m)
                return 0

            stream(p2body)
            b2, _ = _scan_asc(hist, _NBINS2, k2, lane)

            b2u = plsc.bitcast(b2, jnp.uint32)
            fbits = (b1u << 19) | (((b2u << 6) ^ negmask) & jnp.uint32(0x7FFFF))
            thrb[...] = plsc.bitcast(fbits, jnp.float32)
            pltpu.sync_copy(thrb, out_hbm.at[row])

    return thr_kernel


def _mask_body(x_ref, thr_ref, out_ref):
    t = thr_ref[0, 0, 0]
    out_ref[0, 0] = (x_ref[0] >= t).astype(jnp.float32)


def kernel(importance):
    b, h, w = importance.shape
    k = max(1, int(0.75 * h * w))
    thr = _make_threshold_kernel()(importance.reshape(b, h * w))
    mask = pl.pallas_call(
        _mask_body,
        grid=(b,),
        in_specs=[
            pl.BlockSpec((1, h, w), lambda i: (i, 0, 0)),
            pl.BlockSpec((1, 1, 16), lambda i: (i, 0, 0)),
        ],
        out_specs=pl.BlockSpec((1, 1, h, w), lambda i: (i, 0, 0, 0)),
        out_shape=jax.ShapeDtypeStruct((b, 1, h, w), jnp.float32),
    )(importance, thr.reshape(b, 1, 16))
    return (mask, jnp.float32(k / (h * w)))
